# baseline (speedup 1.0000x reference)
import functools

import jax
import jax.numpy as jnp
from jax import lax
from jax.experimental import pallas as pl
from jax.experimental.pallas import tpu as pltpu
from jax.experimental.pallas import tpu_sc as plsc

N = 10000
E = 320000
D = 128
H = 128
NW = 32
CH = 64
NCH0 = 156
NCH1 = 158
G0 = NCH0 // 2
G1 = NCH1 // 2
DW = 16
SLAB = 626
ROWS = SLAB * 16
BLK = 1000
GRID = N // BLK

_SC_MESH = dict(core_axis_name="c", subcore_axis_name="s")
_SC_PARAMS = pltpu.CompilerParams(use_tc_tiling_on_sc=False)


def _sc_segment_sum(h, srcs, dsts, zeros):
    @functools.partial(
        pl.kernel,
        mesh=plsc.VectorSubcoreMesh(**_SC_MESH),
        compiler_params=_SC_PARAMS,
        out_type=jax.ShapeDtypeStruct((2, ROWS, H), jnp.float32),
        scratch_types=[
            pltpu.VMEM((NCH1, CH), jnp.int32),
            pltpu.VMEM((NCH1, CH), jnp.int32),
            pltpu.VMEM((CH, H), jnp.float32),
            pltpu.VMEM((CH, H), jnp.float32),
            pltpu.VMEM_SHARED((ROWS, H), jnp.float32),
            pltpu.SemaphoreType.DMA,
            pltpu.SemaphoreType.DMA,
        ],
    )
    def k(h_ref, src_ref, dst_ref, z_ref, out_ref,
          src_v, dst_v, r0, r1, acc, ss0, ss1):
        rows = (r0, r1)
        ssem = (ss0, ss1)
        cid = lax.axis_index("c")
        sid = lax.axis_index("s")
        wid = cid * 16 + sid
        pltpu.sync_copy(src_ref.at[wid], src_v)
        pltpu.sync_copy(dst_ref.at[wid], dst_v)
        pltpu.sync_copy(z_ref.at[pl.ds(sid * SLAB, SLAB)],
                        acc.at[pl.ds(sid * SLAB, SLAB)])
        plsc.subcore_barrier()

        def s_start(j, b):
            pltpu.async_copy(rows[b], acc.at[dst_v.at[j]], ssem[b],
                             add=True)

        def s_wait(j, b):
            pltpu.make_async_copy(rows[b], acc.at[dst_v.at[j]],
                                  ssem[b]).wait()

        def chunk(j, b, head=False):
            if not head:
                s_wait(j - 2, b)
            pltpu.sync_copy(h_ref.at[src_v.at[j]], rows[b])
            s_start(j, b)

        chunk(0, 0, head=True)
        chunk(1, 1, head=True)

        def body(g, carry):
            @pl.when((cid == 1) | (g < G0))
            def _():
                chunk(2 * g, 0)
                chunk(2 * g + 1, 1)
            return carry

        lax.fori_loop(1, G1, body, 0)
        s_wait(0, 0)
        s_wait(1, 1)

        plsc.subcore_barrier()
        pltpu.sync_copy(acc.at[pl.ds(sid * SLAB, SLAB)],
                        out_ref.at[cid, pl.ds(sid * SLAB, SLAB)])

    return k(h, srcs, dsts, zeros)


def _sc_degree(dsts, ones, zeros):
    @functools.partial(
        pl.kernel,
        mesh=plsc.VectorSubcoreMesh(**_SC_MESH),
        compiler_params=_SC_PARAMS,
        out_type=jax.ShapeDtypeStruct((2, ROWS, DW), jnp.float32),
        scratch_types=[
            pltpu.VMEM((NCH1, CH), jnp.int32),
            pltpu.VMEM((CH, DW), jnp.float32),
            pltpu.VMEM_SHARED((ROWS, DW), jnp.float32),
            pltpu.SemaphoreType.DMA,
            pltpu.SemaphoreType.DMA,
        ],
    )
    def k(dst_ref, ones_ref, z_ref, out_ref, dst_v, ones_v, acc, ss0, ss1):
        ssem = (ss0, ss1)
        cid = lax.axis_index("c")
        sid = lax.axis_index("s")
        wid = cid * 16 + sid
        pltpu.sync_copy(dst_ref.at[wid], dst_v)
        pltpu.sync_copy(ones_ref, ones_v)
        pltpu.sync_copy(z_ref.at[pl.ds(sid * SLAB, SLAB)],
                        acc.at[pl.ds(sid * SLAB, SLAB)])
        plsc.subcore_barrier()

        def s_start(j, b):
            pltpu.async_copy(ones_v, acc.at[dst_v.at[j]], ssem[b],
                             add=True)

        def s_wait(j, b):
            pltpu.make_async_copy(ones_v, acc.at[dst_v.at[j]],
                                  ssem[b]).wait()

        s_start(0, 0)
        s_start(1, 1)

        def body(g, carry):
            @pl.when((cid == 1) | (g < G0))
            def _():
                for q in (0, 1):
                    j = 2 * g + q
                    s_wait(j - 2, q)
                    s_start(j, q)
            return carry

        lax.fori_loop(1, G1, body, 0)
        s_wait(0, 0)
        s_wait(1, 1)

        plsc.subcore_barrier()
        pltpu.sync_copy(acc.at[pl.ds(sid * SLAB, SLAB)],
                        out_ref.at[cid, pl.ds(sid * SLAB, SLAB)])

    return k(dsts, ones, zeros)


def _ln_relu(y, g, b):
    mu = jnp.mean(y, axis=-1, keepdims=True)
    var = jnp.mean((y - mu) ** 2, axis=-1, keepdims=True)
    return jnp.maximum(g * (y - mu) * lax.rsqrt(var + 1e-5) + b, 0.0)


def _tc_layer0(p, pd, x, wlT, bl, wrT, g, b):
    def body(p_ref, pd_ref, x_ref, wl_ref, bl_ref, wr_ref, g_ref, b_ref,
             h_ref, inv_ref):
        agg = p_ref[0] + p_ref[1]
        deg = pd_ref[0, :, 0:1] + pd_ref[1, :, 0:1]
        inv = 1.0 / jnp.maximum(deg, 1.0)
        y = jnp.dot(agg * inv, wl_ref[...],
                    preferred_element_type=jnp.float32)
        y = y + bl_ref[...] + jnp.dot(x_ref[...], wr_ref[...],
                                      preferred_element_type=jnp.float32)
        h_ref[...] = _ln_relu(y, g_ref[...], b_ref[...])
        inv_ref[...] = jnp.broadcast_to(inv, (BLK, 8))

    return pl.pallas_call(
        body,
        grid=(GRID,),
        in_specs=[
            pl.BlockSpec((2, BLK, H), lambda i: (0, i, 0)),
            pl.BlockSpec((2, BLK, DW), lambda i: (0, i, 0)),
            pl.BlockSpec((BLK, D), lambda i: (i, 0)),
            pl.BlockSpec((D, H), lambda i: (0, 0)),
            pl.BlockSpec((1, H), lambda i: (0, 0)),
            pl.BlockSpec((D, H), lambda i: (0, 0)),
            pl.BlockSpec((1, H), lambda i: (0, 0)),
            pl.BlockSpec((1, H), lambda i: (0, 0)),
        ],
        out_specs=[pl.BlockSpec((BLK, H), lambda i: (i, 0)),
                   pl.BlockSpec((BLK, 8), lambda i: (i, 0))],
        out_shape=[jax.ShapeDtypeStruct((N, H), jnp.float32),
                   jax.ShapeDtypeStruct((N, 8), jnp.float32)],
    )(p, pd, x, wlT, bl, wrT, g, b)


def _tc_mid(p, h, invd, wlT, bl, wrT, g, b):
    def body(p_ref, h_ref, inv_ref, wl_ref, bl_ref, wr_ref, g_ref, b_ref,
             o_ref):
        agg = (p_ref[0] + p_ref[1]) * inv_ref[:, 0:1]
        y = jnp.dot(agg, wl_ref[...], preferred_element_type=jnp.float32)
        y = y + bl_ref[...] + jnp.dot(h_ref[...], wr_ref[...],
                                      preferred_element_type=jnp.float32)
        o_ref[...] = _ln_relu(y, g_ref[...], b_ref[...])

    return pl.pallas_call(
        body,
        grid=(GRID,),
        in_specs=[
            pl.BlockSpec((2, BLK, H), lambda i: (0, i, 0)),
            pl.BlockSpec((BLK, H), lambda i: (i, 0)),
            pl.BlockSpec((BLK, 8), lambda i: (i, 0)),
            pl.BlockSpec((H, H), lambda i: (0, 0)),
            pl.BlockSpec((1, H), lambda i: (0, 0)),
            pl.BlockSpec((H, H), lambda i: (0, 0)),
            pl.BlockSpec((1, H), lambda i: (0, 0)),
            pl.BlockSpec((1, H), lambda i: (0, 0)),
        ],
        out_specs=pl.BlockSpec((BLK, H), lambda i: (i, 0)),
        out_shape=jax.ShapeDtypeStruct((N, H), jnp.float32),
    )(p, h, invd, wlT, bl, wrT, g, b)


def _tc_final(p, h, invd, wlT, bl, wrT, g, b, w1T, b1, w2T, b2, w3T, b3):
    def body(p_ref, h_ref, inv_ref, wl_ref, bl_ref, wr_ref, g_ref, b_ref,
             w1_ref, b1_ref, w2_ref, b2_ref, w3_ref, b3_ref, o_ref):
        agg = (p_ref[0] + p_ref[1]) * inv_ref[:, 0:1]
        y = jnp.dot(agg, wl_ref[...], preferred_element_type=jnp.float32)
        y = y + bl_ref[...] + jnp.dot(h_ref[...], wr_ref[...],
                                      preferred_element_type=jnp.float32)
        t = _ln_relu(y, g_ref[...], b_ref[...])
        t = jnp.maximum(jnp.dot(t, w1_ref[...],
                                preferred_element_type=jnp.float32)
                        + b1_ref[...], 0.0)
        t = jnp.maximum(jnp.dot(t, w2_ref[...],
                                preferred_element_type=jnp.float32)
                        + b2_ref[...], 0.0)
        o_ref[...] = jnp.dot(t, w3_ref[...],
                             preferred_element_type=jnp.float32) + b3_ref[...]

    return pl.pallas_call(
        body,
        grid=(GRID,),
        in_specs=[
            pl.BlockSpec((2, BLK, H), lambda i: (0, i, 0)),
            pl.BlockSpec((BLK, H), lambda i: (i, 0)),
            pl.BlockSpec((BLK, 8), lambda i: (i, 0)),
            pl.BlockSpec((H, H), lambda i: (0, 0)),
            pl.BlockSpec((1, H), lambda i: (0, 0)),
            pl.BlockSpec((H, H), lambda i: (0, 0)),
            pl.BlockSpec((1, H), lambda i: (0, 0)),
            pl.BlockSpec((1, H), lambda i: (0, 0)),
            pl.BlockSpec((H, H // 2), lambda i: (0, 0)),
            pl.BlockSpec((1, H // 2), lambda i: (0, 0)),
            pl.BlockSpec((H // 2, H // 4), lambda i: (0, 0)),
            pl.BlockSpec((1, H // 4), lambda i: (0, 0)),
            pl.BlockSpec((H // 4, 8), lambda i: (0, 0)),
            pl.BlockSpec((1, 8), lambda i: (0, 0)),
        ],
        out_specs=pl.BlockSpec((BLK, 8), lambda i: (i, 0)),
        out_shape=jax.ShapeDtypeStruct((N, 8), jnp.float32),
    )(p, h, invd, wlT, bl, wrT, g, b, w1T, b1, w2T, b2, w3T, b3)


def kernel(x, edge_index, conv0_Wl, conv0_bl, conv0_Wr, norm0_g, norm0_b,
           conv1_Wl, conv1_bl, conv1_Wr, norm1_g, norm1_b,
           conv2_Wl, conv2_bl, conv2_Wr, norm2_g, norm2_b,
           reg_W1, reg_b1, reg_W2, reg_b2, reg_W3, reg_b3):
    src = edge_index[0]
    dst = edge_index[1]

    pad = 16 * (NCH0 + NCH1) * CH - E

    def split(a):
        e0 = 16 * NCH0 * CH
        a0 = a[:e0].reshape(16, NCH0, CH)
        a0 = jnp.pad(a0, ((0, 0), (0, NCH1 - NCH0), (0, 0)))
        a1 = a[e0:].reshape(16, NCH1, CH)
        return jnp.concatenate([a0, a1], axis=0)

    srcs = split(jnp.concatenate([src, jnp.zeros((pad,), jnp.int32)]))
    dsts = split(jnp.concatenate([dst, jnp.full((pad,), N, jnp.int32)]))
    z = jnp.zeros((ROWS, H), jnp.float32)
    zd = jnp.zeros((ROWS, DW), jnp.float32)
    ones = jnp.ones((CH, DW), jnp.float32)

    pd = _sc_degree(dsts, ones, zd)
    p0 = _sc_segment_sum(x, srcs, dsts, z)
    h1, invd = _tc_layer0(p0, pd, x, conv0_Wl.T, conv0_bl.reshape(1, H),
                          conv0_Wr.T, norm0_g.reshape(1, H),
                          norm0_b.reshape(1, H))
    p1 = _sc_segment_sum(h1, srcs, dsts, z)
    h2 = _tc_mid(p1, h1, invd, conv1_Wl.T, conv1_bl.reshape(1, H),
                 conv1_Wr.T, norm1_g.reshape(1, H), norm1_b.reshape(1, H))
    p2 = _sc_segment_sum(h2, srcs, dsts, z)
    out = _tc_final(p2, h2, invd, conv2_Wl.T, conv2_bl.reshape(1, H),
                    conv2_Wr.T, norm2_g.reshape(1, H), norm2_b.reshape(1, H),
                    reg_W1.T, reg_b1.reshape(1, H // 2),
                    reg_W2.T, reg_b2.reshape(1, H // 4),
                    reg_W3.T, reg_b3.reshape(1, 8))
    return out


# speedup vs baseline: 1.4199x; 1.4199x over previous
import functools

import jax
import jax.numpy as jnp
from jax import lax
from jax.experimental import pallas as pl
from jax.experimental.pallas import tpu as pltpu
from jax.experimental.pallas import tpu_sc as plsc

N = 10000
E = 320000
D = 128
H = 128
NW = 32
CH = 80
NCH0 = 124
NCH1 = 126
G0 = NCH0 // 2
G1 = NCH1 // 2
DW = 16
SLAB = 626
ROWS = SLAB * 16
BLK = 1000
GRID = N // BLK

_SC_MESH = dict(core_axis_name="c", subcore_axis_name="s")
_SC_PARAMS = pltpu.CompilerParams(use_tc_tiling_on_sc=False)


def _sc_segment_sum(h, srcs, dsts, zeros):
    @functools.partial(
        pl.kernel,
        mesh=plsc.VectorSubcoreMesh(**_SC_MESH),
        compiler_params=_SC_PARAMS,
        out_type=jax.ShapeDtypeStruct((2, ROWS, H), jnp.float32),
        scratch_types=[
            pltpu.VMEM((NCH1, CH), jnp.int32),
            pltpu.VMEM((NCH1, CH), jnp.int32),
            pltpu.VMEM((CH, H), jnp.float32),
            pltpu.VMEM((CH, H), jnp.float32),
            pltpu.VMEM_SHARED((ROWS, H), jnp.float32),
            pltpu.SemaphoreType.DMA,
            pltpu.SemaphoreType.DMA,
        ],
    )
    def k(h_ref, src_ref, dst_ref, z_ref, out_ref,
          src_v, dst_v, r0, r1, acc, ss0, ss1):
        rows = (r0, r1)
        ssem = (ss0, ss1)
        cid = lax.axis_index("c")
        sid = lax.axis_index("s")
        wid = cid * 16 + sid
        pltpu.sync_copy(src_ref.at[wid], src_v)
        pltpu.sync_copy(dst_ref.at[wid], dst_v)
        pltpu.sync_copy(z_ref.at[pl.ds(sid * SLAB, SLAB)],
                        acc.at[pl.ds(sid * SLAB, SLAB)])
        plsc.subcore_barrier()

        def s_start(j, b):
            pltpu.async_copy(rows[b], acc.at[dst_v.at[j]], ssem[b],
                             add=True)

        def s_wait(j, b):
            pltpu.make_async_copy(rows[b], acc.at[dst_v.at[j]],
                                  ssem[b]).wait()

        def chunk(j, b, head=False):
            if not head:
                s_wait(j - 2, b)
            pltpu.sync_copy(h_ref.at[src_v.at[j]], rows[b])
            s_start(j, b)

        chunk(0, 0, head=True)
        chunk(1, 1, head=True)

        def body(g, carry):
            @pl.when((cid == 1) | (g < G0))
            def _():
                chunk(2 * g, 0)
                chunk(2 * g + 1, 1)
            return carry

        lax.fori_loop(1, G1, body, 0)
        s_wait(0, 0)
        s_wait(1, 1)

        plsc.subcore_barrier()
        pltpu.sync_copy(acc.at[pl.ds(sid * SLAB, SLAB)],
                        out_ref.at[cid, pl.ds(sid * SLAB, SLAB)])

    return k(h, srcs, dsts, zeros)


def _sc_degree(dsts, ones, zeros):
    @functools.partial(
        pl.kernel,
        mesh=plsc.VectorSubcoreMesh(**_SC_MESH),
        compiler_params=_SC_PARAMS,
        out_type=jax.ShapeDtypeStruct((2, ROWS, DW), jnp.float32),
        scratch_types=[
            pltpu.VMEM((NCH1, CH), jnp.int32),
            pltpu.VMEM((CH, DW), jnp.float32),
            pltpu.VMEM_SHARED((ROWS, DW), jnp.float32),
            pltpu.SemaphoreType.DMA,
            pltpu.SemaphoreType.DMA,
        ],
    )
    def k(dst_ref, ones_ref, z_ref, out_ref, dst_v, ones_v, acc, ss0, ss1):
        ssem = (ss0, ss1)
        cid = lax.axis_index("c")
        sid = lax.axis_index("s")
        wid = cid * 16 + sid
        pltpu.sync_copy(dst_ref.at[wid], dst_v)
        pltpu.sync_copy(ones_ref, ones_v)
        pltpu.sync_copy(z_ref.at[pl.ds(sid * SLAB, SLAB)],
                        acc.at[pl.ds(sid * SLAB, SLAB)])
        plsc.subcore_barrier()

        def s_start(j, b):
            pltpu.async_copy(ones_v, acc.at[dst_v.at[j]], ssem[b],
                             add=True)

        def s_wait(j, b):
            pltpu.make_async_copy(ones_v, acc.at[dst_v.at[j]],
                                  ssem[b]).wait()

        s_start(0, 0)
        s_start(1, 1)

        def body(g, carry):
            @pl.when((cid == 1) | (g < G0))
            def _():
                for q in (0, 1):
                    j = 2 * g + q
                    s_wait(j - 2, q)
                    s_start(j, q)
            return carry

        lax.fori_loop(1, G1, body, 0)
        s_wait(0, 0)
        s_wait(1, 1)

        plsc.subcore_barrier()
        pltpu.sync_copy(acc.at[pl.ds(sid * SLAB, SLAB)],
                        out_ref.at[cid, pl.ds(sid * SLAB, SLAB)])

    return k(dsts, ones, zeros)


def _ln_relu(y, g, b):
    mu = jnp.mean(y, axis=-1, keepdims=True)
    var = jnp.mean((y - mu) ** 2, axis=-1, keepdims=True)
    return jnp.maximum(g * (y - mu) * lax.rsqrt(var + 1e-5) + b, 0.0)


def _tc_layer0(p, pd, x, wlT, bl, wrT, g, b):
    def body(p_ref, pd_ref, x_ref, wl_ref, bl_ref, wr_ref, g_ref, b_ref,
             h_ref, inv_ref):
        agg = p_ref[0] + p_ref[1]
        deg = pd_ref[0, :, 0:1] + pd_ref[1, :, 0:1]
        inv = 1.0 / jnp.maximum(deg, 1.0)
        y = jnp.dot(agg * inv, wl_ref[...],
                    preferred_element_type=jnp.float32)
        y = y + bl_ref[...] + jnp.dot(x_ref[...], wr_ref[...],
                                      preferred_element_type=jnp.float32)
        h_ref[...] = _ln_relu(y, g_ref[...], b_ref[...])
        inv_ref[...] = jnp.broadcast_to(inv, (BLK, 8))

    return pl.pallas_call(
        body,
        grid=(GRID,),
        in_specs=[
            pl.BlockSpec((2, BLK, H), lambda i: (0, i, 0)),
            pl.BlockSpec((2, BLK, DW), lambda i: (0, i, 0)),
            pl.BlockSpec((BLK, D), lambda i: (i, 0)),
            pl.BlockSpec((D, H), lambda i: (0, 0)),
            pl.BlockSpec((1, H), lambda i: (0, 0)),
            pl.BlockSpec((D, H), lambda i: (0, 0)),
            pl.BlockSpec((1, H), lambda i: (0, 0)),
            pl.BlockSpec((1, H), lambda i: (0, 0)),
        ],
        out_specs=[pl.BlockSpec((BLK, H), lambda i: (i, 0)),
                   pl.BlockSpec((BLK, 8), lambda i: (i, 0))],
        out_shape=[jax.ShapeDtypeStruct((N, H), jnp.float32),
                   jax.ShapeDtypeStruct((N, 8), jnp.float32)],
    )(p, pd, x, wlT, bl, wrT, g, b)


def _tc_mid(p, h, invd, wlT, bl, wrT, g, b):
    def body(p_ref, h_ref, inv_ref, wl_ref, bl_ref, wr_ref, g_ref, b_ref,
             o_ref):
        agg = (p_ref[0] + p_ref[1]) * inv_ref[:, 0:1]
        y = jnp.dot(agg, wl_ref[...], preferred_element_type=jnp.float32)
        y = y + bl_ref[...] + jnp.dot(h_ref[...], wr_ref[...],
                                      preferred_element_type=jnp.float32)
        o_ref[...] = _ln_relu(y, g_ref[...], b_ref[...])

    return pl.pallas_call(
        body,
        grid=(GRID,),
        in_specs=[
            pl.BlockSpec((2, BLK, H), lambda i: (0, i, 0)),
            pl.BlockSpec((BLK, H), lambda i: (i, 0)),
            pl.BlockSpec((BLK, 8), lambda i: (i, 0)),
            pl.BlockSpec((H, H), lambda i: (0, 0)),
            pl.BlockSpec((1, H), lambda i: (0, 0)),
            pl.BlockSpec((H, H), lambda i: (0, 0)),
            pl.BlockSpec((1, H), lambda i: (0, 0)),
            pl.BlockSpec((1, H), lambda i: (0, 0)),
        ],
        out_specs=pl.BlockSpec((BLK, H), lambda i: (i, 0)),
        out_shape=jax.ShapeDtypeStruct((N, H), jnp.float32),
    )(p, h, invd, wlT, bl, wrT, g, b)


def _tc_final(p, h, invd, wlT, bl, wrT, g, b, w1T, b1, w2T, b2, w3T, b3):
    def body(p_ref, h_ref, inv_ref, wl_ref, bl_ref, wr_ref, g_ref, b_ref,
             w1_ref, b1_ref, w2_ref, b2_ref, w3_ref, b3_ref, o_ref):
        agg = (p_ref[0] + p_ref[1]) * inv_ref[:, 0:1]
        y = jnp.dot(agg, wl_ref[...], preferred_element_type=jnp.float32)
        y = y + bl_ref[...] + jnp.dot(h_ref[...], wr_ref[...],
                                      preferred_element_type=jnp.float32)
        t = _ln_relu(y, g_ref[...], b_ref[...])
        t = jnp.maximum(jnp.dot(t, w1_ref[...],
                                preferred_element_type=jnp.float32)
                        + b1_ref[...], 0.0)
        t = jnp.maximum(jnp.dot(t, w2_ref[...],
                                preferred_element_type=jnp.float32)
                        + b2_ref[...], 0.0)
        o_ref[...] = jnp.dot(t, w3_ref[...],
                             preferred_element_type=jnp.float32) + b3_ref[...]

    return pl.pallas_call(
        body,
        grid=(GRID,),
        in_specs=[
            pl.BlockSpec((2, BLK, H), lambda i: (0, i, 0)),
            pl.BlockSpec((BLK, H), lambda i: (i, 0)),
            pl.BlockSpec((BLK, 8), lambda i: (i, 0)),
            pl.BlockSpec((H, H), lambda i: (0, 0)),
            pl.BlockSpec((1, H), lambda i: (0, 0)),
            pl.BlockSpec((H, H), lambda i: (0, 0)),
            pl.BlockSpec((1, H), lambda i: (0, 0)),
            pl.BlockSpec((1, H), lambda i: (0, 0)),
            pl.BlockSpec((H, H // 2), lambda i: (0, 0)),
            pl.BlockSpec((1, H // 2), lambda i: (0, 0)),
            pl.BlockSpec((H // 2, H // 4), lambda i: (0, 0)),
            pl.BlockSpec((1, H // 4), lambda i: (0, 0)),
            pl.BlockSpec((H // 4, 8), lambda i: (0, 0)),
            pl.BlockSpec((1, 8), lambda i: (0, 0)),
        ],
        out_specs=pl.BlockSpec((BLK, 8), lambda i: (i, 0)),
        out_shape=jax.ShapeDtypeStruct((N, 8), jnp.float32),
    )(p, h, invd, wlT, bl, wrT, g, b, w1T, b1, w2T, b2, w3T, b3)


def kernel(x, edge_index, conv0_Wl, conv0_bl, conv0_Wr, norm0_g, norm0_b,
           conv1_Wl, conv1_bl, conv1_Wr, norm1_g, norm1_b,
           conv2_Wl, conv2_bl, conv2_Wr, norm2_g, norm2_b,
           reg_W1, reg_b1, reg_W2, reg_b2, reg_W3, reg_b3):
    src = edge_index[0]
    dst = edge_index[1]

    pad = 16 * (NCH0 + NCH1) * CH - E

    def split(a):
        e0 = 16 * NCH0 * CH
        a0 = a[:e0].reshape(16, NCH0, CH)
        a0 = jnp.pad(a0, ((0, 0), (0, NCH1 - NCH0), (0, 0)))
        a1 = a[e0:].reshape(16, NCH1, CH)
        return jnp.concatenate([a0, a1], axis=0)

    srcs = split(jnp.concatenate([src, jnp.zeros((pad,), jnp.int32)]))
    dsts = split(jnp.concatenate([dst, jnp.full((pad,), N, jnp.int32)]))
    z = jnp.zeros((ROWS, H), jnp.float32)
    zd = jnp.zeros((ROWS, DW), jnp.float32)
    ones = jnp.ones((CH, DW), jnp.float32)

    pd = _sc_degree(dsts, ones, zd)
    p0 = _sc_segment_sum(x, srcs, dsts, z)
    h1, invd = _tc_layer0(p0, pd, x, conv0_Wl.T, conv0_bl.reshape(1, H),
                          conv0_Wr.T, norm0_g.reshape(1, H),
                          norm0_b.reshape(1, H))
    p1 = _sc_segment_sum(h1, srcs, dsts, z)
    h2 = _tc_mid(p1, h1, invd, conv1_Wl.T, conv1_bl.reshape(1, H),
                 conv1_Wr.T, norm1_g.reshape(1, H), norm1_b.reshape(1, H))
    p2 = _sc_segment_sum(h2, srcs, dsts, z)
    out = _tc_final(p2, h2, invd, conv2_Wl.T, conv2_bl.reshape(1, H),
                    conv2_Wr.T, norm2_g.reshape(1, H), norm2_b.reshape(1, H),
                    reg_W1.T, reg_b1.reshape(1, H // 2),
                    reg_W2.T, reg_b2.reshape(1, H // 4),
                    reg_W3.T, reg_b3.reshape(1, 8))
    return out
